# baseline (speedup 1.0000x reference)
import functools

import jax
import jax.numpy as jnp
from jax import lax
from jax.experimental import pallas as pl
from jax.experimental.pallas import tpu as pltpu
from jax.experimental.pallas import tpu_sc as plsc

_N = 10000
_E = 160000
_D = 256
_NUM_LAYERS = 47
_ELTWISE = {3, 5, 7, 9, 11, 13, 19, 21, 23, 25, 27, 29, 35, 37, 39, 41, 43, 45}
_CONCAT = {15, 31}

_NC = 2
_NS = 16
_G = 128
_EPAD = 163840
_ECH = _EPAD // _G
_CT = _ECH // _NS
_NPAD = 10240
_RT = _NPAD // _NS
_RC = 64
_NWB = _RT // _RC

_P = 20000
_PPAD = 20480
_PG = 128
_PCH = _PPAD // _PG
_NW = _NC * _NS
_PWC = _PCH // _NW

_mesh = plsc.VectorSubcoreMesh(core_axis_name="c", subcore_axis_name="s")


@functools.partial(
    pl.kernel,
    mesh=_mesh,
    out_type=(
        jax.ShapeDtypeStruct((_NPAD, 128), jnp.float32),
        jax.ShapeDtypeStruct((_NPAD, 128), jnp.float32),
    ),
    scratch_types=[
        pltpu.VMEM((_CT, _G), jnp.int32),
        pltpu.VMEM((_CT, _G), jnp.int32),
        pltpu.VMEM((_G, 128), jnp.float32),
        pltpu.VMEM((_RC, 128), jnp.float32),
        pltpu.VMEM((_G,), jnp.int32),
        pltpu.VMEM((_G,), jnp.int32),
        pltpu.VMEM_SHARED((_NPAD, 128), jnp.float32),
        pltpu.SemaphoreType.DMA,
    ],
)
def _spmm_sc(p0_hbm, p1_hbm, src_hbm, dst_hbm, m0_hbm, m1_hbm,
             src_v, dst_v, rows_v, buf_v, src_cur, dst_cur, acc_sh, sem):
    c = lax.axis_index("c")
    s = lax.axis_index("s")

    def _zrow(r, carry):
        for q in range(8):
            buf_v[r, pl.ds(16 * q, 16)] = jnp.zeros((16,), jnp.float32)
        return carry
    lax.fori_loop(0, _RC, _zrow, 0)
    row0 = s * _RT
    for k in range(_NWB):
        pltpu.sync_copy(buf_v, acc_sh.at[pl.ds(row0 + k * _RC, _RC)])
    plsc.subcore_barrier()

    pltpu.sync_copy(src_hbm.at[pl.ds(s * _CT, _CT)], src_v)
    pltpu.sync_copy(dst_hbm.at[pl.ds(s * _CT, _CT)], dst_v)

    def _edges(p_hbm):
        def _body(j, carry):
            for q in range(_G // 16):
                src_cur[pl.ds(16 * q, 16)] = src_v[j, pl.ds(16 * q, 16)]
                dst_cur[pl.ds(16 * q, 16)] = dst_v[j, pl.ds(16 * q, 16)]
            pltpu.async_copy(p_hbm.at[src_cur], rows_v, sem).wait()
            pltpu.sync_copy(rows_v, acc_sh.at[dst_cur], add=True)
            return carry
        lax.fori_loop(0, _CT, _body, 0)

    @pl.when(c == 0)
    def _():
        _edges(p0_hbm)

    @pl.when(c == 1)
    def _():
        _edges(p1_hbm)

    plsc.subcore_barrier()

    def _writeback(m_hbm):
        for k in range(_NWB):
            r0 = row0 + k * _RC
            pltpu.sync_copy(acc_sh.at[pl.ds(r0, _RC)], buf_v)
            pltpu.sync_copy(buf_v, m_hbm.at[pl.ds(r0, _RC)])

    @pl.when(c == 0)
    def _():
        _writeback(m0_hbm)

    @pl.when(c == 1)
    def _():
        _writeback(m1_hbm)


@functools.partial(
    pl.kernel,
    mesh=_mesh,
    out_type=jax.ShapeDtypeStruct((_NPAD, 16), jnp.float32),
    scratch_types=[
        pltpu.VMEM((_CT, _G), jnp.int32),
        pltpu.VMEM((_G, 16), jnp.float32),
        pltpu.VMEM((_RT, 16), jnp.float32),
        pltpu.VMEM((_G,), jnp.int32),
        pltpu.VMEM_SHARED((_NPAD, 16), jnp.float32),
        pltpu.SemaphoreType.DMA,
    ],
)
def _deg_sc(dst_hbm, deg_hbm, dst_v, ones_v, tbuf_v, dst_cur, acc_sh, sem):
    s = lax.axis_index("s")

    def _orow(r, carry):
        ones_v[r, :] = jnp.ones((16,), jnp.float32)
        return carry
    lax.fori_loop(0, _G, _orow, 0)

    def _zrow(r, carry):
        tbuf_v[r, :] = jnp.zeros((16,), jnp.float32)
        return carry
    lax.fori_loop(0, _RT, _zrow, 0)

    if True:
        pltpu.sync_copy(tbuf_v, acc_sh.at[pl.ds(s * _RT, _RT)])
        plsc.subcore_barrier()

        pltpu.sync_copy(dst_hbm.at[s], dst_v)

        def _body(j, carry):
            for q in range(_G // 16):
                dst_cur[pl.ds(16 * q, 16)] = dst_v[j, pl.ds(16 * q, 16)]
            pltpu.sync_copy(ones_v, acc_sh.at[dst_cur], add=True)
            return carry
        lax.fori_loop(0, _CT, _body, 0)

        plsc.subcore_barrier()
        pltpu.sync_copy(acc_sh.at[pl.ds(s * _RT, _RT)], tbuf_v)
        pltpu.sync_copy(tbuf_v, deg_hbm.at[pl.ds(s * _RT, _RT)])


@functools.partial(
    pl.kernel,
    mesh=_mesh,
    out_type=jax.ShapeDtypeStruct((_PPAD, _D), jnp.float32),
    scratch_types=[
        pltpu.VMEM((_PCH, _PG), jnp.int32),
        pltpu.VMEM((_PCH, _PG), jnp.int32),
        pltpu.VMEM((_PG, _D), jnp.float32),
        pltpu.VMEM((_PG, _D), jnp.float32),
        pltpu.VMEM((_PG,), jnp.int32),
        pltpu.VMEM((_PG,), jnp.int32),
        pltpu.SemaphoreType.DMA,
    ],
)
def _unpool_sc(h_hbm, pi0_hbm, pi1_hbm, out_hbm, pi0_v, pi1_v, b0_v, b1_v,
               pc0, pc1, sem):
    c = lax.axis_index("c")
    s = lax.axis_index("s")
    wid = s * _NC + c
    pltpu.sync_copy(pi0_hbm, pi0_v)
    pltpu.sync_copy(pi1_hbm, pi1_v)
    for j in range(_PWC):
        for q in range(_PG // 16):
            pc0[pl.ds(16 * q, 16)] = pi0_v[wid * _PWC + j, pl.ds(16 * q, 16)]
            pc1[pl.ds(16 * q, 16)] = pi1_v[wid * _PWC + j, pl.ds(16 * q, 16)]
        pltpu.async_copy(h_hbm.at[pc0], b0_v, sem).wait()
        pltpu.async_copy(h_hbm.at[pc1], b1_v, sem).wait()

        def _arow(r, carry):
            for q in range(_D // 16):
                a = b0_v[r, pl.ds(16 * q, 16)]
                b = b1_v[r, pl.ds(16 * q, 16)]
                b0_v[r, pl.ds(16 * q, 16)] = (a + b) * 0.5
            return carry
        lax.fori_loop(0, _PG, _arow, 0)
        pltpu.sync_copy(b0_v, out_hbm.at[pl.ds(wid * _PWC * _PG + j * _PG, _PG)])


_RB = 1000


def _mm_body(h_ref, w_ref, zl_ref, p0_ref, p1_ref):
    z = jnp.dot(h_ref[...], w_ref[...], preferred_element_type=jnp.float32)
    zl_ref[...] = z[:, :_D]
    p0_ref[...] = z[:, _D:_D + 128]
    p1_ref[...] = z[:, _D + 128:]


def _matmul(h, w):
    n, k = h.shape
    return pl.pallas_call(
        _mm_body,
        grid=(n // _RB,),
        in_specs=[
            pl.BlockSpec((_RB, k), lambda i: (i, 0)),
            pl.BlockSpec((k, 2 * _D), lambda i: (0, 0)),
        ],
        out_specs=[
            pl.BlockSpec((_RB, _D), lambda i: (i, 0)),
            pl.BlockSpec((_RB, 128), lambda i: (i, 0)),
            pl.BlockSpec((_RB, 128), lambda i: (i, 0)),
        ],
        out_shape=[
            jax.ShapeDtypeStruct((n, _D), jnp.float32),
            jax.ShapeDtypeStruct((n, 128), jnp.float32),
            jax.ShapeDtypeStruct((n, 128), jnp.float32),
        ],
    )(h, w)


def _ew_body(zl_ref, m0_ref, m1_ref, dv_ref, b_ref, *rest, relu, resid):
    if resid:
        res_ref, out_ref = rest
    else:
        (out_ref,) = rest
    dv = dv_ref[...][:, 0:1]
    m = jnp.concatenate([m0_ref[...], m1_ref[...]], axis=1) * dv
    o = zl_ref[...] + m + b_ref[...]
    if relu:
        o = jnp.maximum(o, 0.0)
    if resid:
        o = (o + res_ref[...]) * 0.5
    out_ref[...] = o


def _ew(zl, m0, m1, dv, b, res, relu):
    resid = res is not None
    args = [zl, m0, m1, dv, b] + ([res] if resid else [])
    in_specs = [
        pl.BlockSpec((_RB, _D), lambda i: (i, 0)),
        pl.BlockSpec((_RB, 128), lambda i: (i, 0)),
        pl.BlockSpec((_RB, 128), lambda i: (i, 0)),
        pl.BlockSpec((_RB, 16), lambda i: (i, 0)),
        pl.BlockSpec((1, _D), lambda i: (0, 0)),
    ] + ([pl.BlockSpec((_RB, _D), lambda i: (i, 0))] if resid else [])
    return pl.pallas_call(
        functools.partial(_ew_body, relu=relu, resid=resid),
        grid=(_N // _RB,),
        in_specs=in_specs,
        out_specs=pl.BlockSpec((_RB, _D), lambda i: (i, 0)),
        out_shape=jax.ShapeDtypeStruct((_N, _D), jnp.float32),
    )(*args)


def kernel(x, edge_index, pool_idx1, pool_idx2, Wn, Wl, bs):
    srcf = edge_index[0].astype(jnp.int32)
    dstf = edge_index[1].astype(jnp.int32)
    deg = jax.ops.segment_sum(jnp.ones((_E,), jnp.float32), dstf,
                              num_segments=_N)
    dinv = jnp.tile((1.0 / jnp.clip(deg, 1.0))[:, None], (1, 16))

    acts = [x]
    for i in range(_NUM_LAYERS):
        h = acts[-1]
        wcat = jnp.concatenate([Wl[i], Wn[i]], axis=1)
        zl, p0, p1 = _matmul(h, wcat)
        mfull = jax.ops.segment_sum(
            jnp.concatenate([p0, p1], axis=1)[srcf], dstf, num_segments=_N)
        m0 = mfull[:, :128]
        m1 = mfull[:, 128:]
        b2 = bs[i].reshape(1, _D)
        res = acts[-2] if i in _ELTWISE else None
        hidden = _ew(zl, m0, m1, dinv, b2, res, relu=(i != _NUM_LAYERS - 1))
        if i in _CONCAT:
            hidden = jnp.concatenate([hidden, acts[-2]], axis=-1)
        acts.append(hidden)

    def _unpool(h, pool_idx):
        pi = pool_idx.astype(jnp.int32)
        pad = ((0, _PPAD - _P),)
        pi0 = jnp.pad(pi[:, 0], pad).reshape(_PCH, _PG)
        pi1 = jnp.pad(pi[:, 1], pad).reshape(_PCH, _PG)
        new = _unpool_sc(h, pi0, pi1)
        return jnp.concatenate([h, new[:_P]], axis=0)

    output1 = acts[15]
    output2 = acts[31]
    return (output1, _unpool(output1, pool_idx1), output2,
            _unpool(output2, pool_idx2), acts[-1])
